# baseline (speedup 1.0000x reference)
import functools

import jax
import jax.numpy as jnp
import numpy as np
from jax import lax
from jax.experimental import pallas as pl
from jax.experimental.pallas import tpu as pltpu
from jax.experimental.pallas import tpu_sc as plsc

NUM_USERS = 4000
NUM_ITEMS = 6000
N_NODES = NUM_USERS + NUM_ITEMS
D = 128
E = 320000
B = 1024
LAYERS = 3

NC = 2
NS = 16
N_PAD = N_NODES + 16
HALF = N_PAD // NC
ACC_ROWS = HALF + 16
TRASH = HALF
ZROWS = 312
ZREM = ACC_ROWS - NS * ZROWS
OROWS = 312
OREM = HALF - NS * OROWS
EPW = E // NS
K = 80
NCH = EPW // K
NW = NC * NS
PEPW = E // NW
SEG = 10080
NSEG = NW * SEG

_mesh = plsc.VectorSubcoreMesh(
    core_axis_name="c", subcore_axis_name="s", num_cores=NC, num_subcores=NS
)


ZB = 5040

_STEP = np.asarray([1 if e > k else 0 for k in range(15) for e in range(16)],
                   np.int32)


def _partition_body(src_hbm, dst_hbm, w_hbm, step_hbm, psrc, pdst, pw, pcnt,
                    isrc, idst, iw, posb0, posb1, posb2, posb3,
                    zbi, zbf, cntb, stepb, sem):
    c = lax.axis_index("c")
    s = lax.axis_index("s")
    wid = s * NC + c
    base = wid * PEPW
    pltpu.sync_copy(src_hbm.at[pl.ds(base, PEPW)], isrc)
    pltpu.sync_copy(dst_hbm.at[pl.ds(base, PEPW)], idst)
    pltpu.sync_copy(w_hbm.at[pl.ds(base, PEPW)], iw)
    pltpu.sync_copy(step_hbm, stepb)

    z_i = jnp.zeros((16,), jnp.int32)
    z_f = jnp.zeros((16,), jnp.float32)

    def zro(i, _):
        zbi[pl.ds(i * 16, 16)] = z_i
        zbf[pl.ds(i * 16, 16)] = z_f
        return 0

    lax.fori_loop(0, ZB // 16, zro, 0)
    for half in (0, 1):
        off = wid * SEG + half * ZB
        pltpu.sync_copy(zbi, psrc.at[pl.ds(off, ZB)])
        pltpu.sync_copy(zbf, pw.at[pl.ds(off, ZB)])
        pltpu.sync_copy(zbi, psrc.at[pl.ds(NSEG + off, ZB)])
        pltpu.sync_copy(zbf, pw.at[pl.ds(NSEG + off, ZB)])

    iota = lax.iota(jnp.int32, 16)
    lbase = wid * SEG
    hbase = NSEG + wid * SEG

    posbs = (posb0, posb1, posb2, posb3)

    def compute_issue(cb, pb, lofs, hofs):
        for j in range(K // 16):
            dv = idst[pl.ds(cb + j * 16, 16)]
            islow = -((dv - HALF) >> 31)
            prefix = jnp.zeros((16,), jnp.int32)
            nl = 0
            for k in range(16):
                ilk = -((dv[k] - HALF) >> 31)
                if k < 15:
                    prefix = prefix + stepb[pl.ds(k * 16, 16)] * ilk
                nl = nl + ilk
            poslow = prefix + (lbase + lofs)
            poshigh = (iota - prefix) + (hbase + hofs)
            pos = poshigh + (poslow - poshigh) * islow
            pb[pl.ds(j * 16, 16)] = pos
            lofs = lofs + nl
            hofs = hofs + (16 - nl)
        sl = pl.ds(cb, K)
        pltpu.async_copy(isrc.at[sl], psrc.at[pb], sem)
        pltpu.async_copy(idst.at[sl], pdst.at[pb], sem)
        pltpu.async_copy(iw.at[sl], pw.at[pb], sem)
        return lofs, hofs

    def drain3():
        for _ in range(3):
            pltpu.make_async_copy(
                isrc.at[pl.ds(0, K)], psrc.at[pl.ds(0, K)], sem).wait()

    NCHP = PEPW // K
    lofs, hofs = 0, 0
    for i in range(4):
        lofs, hofs = compute_issue(i * K, posbs[i], lofs, hofs)

    def quad(q, carry):
        lofs, hofs = carry
        for i in range(4):
            drain3()
            lofs, hofs = compute_issue((4 + q * 4 + i) * K, posbs[i],
                                       lofs, hofs)
        return (lofs, hofs)

    lofs, hofs = lax.fori_loop(0, (NCHP - 5) // 4, quad, (lofs, hofs))
    drain3()
    lofs, hofs = compute_issue((NCHP - 1) * K, posbs[0], lofs, hofs)
    for _ in range(4):
        drain3()

    cntb[pl.ds(0, 16)] = jnp.full((16,), lofs, jnp.int32)
    pltpu.sync_copy(cntb, pcnt.at[pl.ds(wid * 16, 16)])
    cntb[pl.ds(0, 16)] = jnp.full((16,), hofs, jnp.int32)
    pltpu.sync_copy(cntb, pcnt.at[pl.ds((NW + wid) * 16, 16)])


_partition = functools.partial(
    pl.kernel,
    out_type=(
        jax.ShapeDtypeStruct((2 * NSEG,), jnp.int32),
        jax.ShapeDtypeStruct((2 * NSEG,), jnp.int32),
        jax.ShapeDtypeStruct((2 * NSEG,), jnp.float32),
        jax.ShapeDtypeStruct((2 * NW * 16,), jnp.int32),
    ),
    mesh=_mesh,
    scratch_types=[
        pltpu.VMEM((PEPW,), jnp.int32),
        pltpu.VMEM((PEPW,), jnp.int32),
        pltpu.VMEM((PEPW,), jnp.float32),
        pltpu.VMEM((K,), jnp.int32),
        pltpu.VMEM((K,), jnp.int32),
        pltpu.VMEM((K,), jnp.int32),
        pltpu.VMEM((K,), jnp.int32),
        pltpu.VMEM((ZB,), jnp.int32),
        pltpu.VMEM((ZB,), jnp.float32),
        pltpu.VMEM((16,), jnp.int32),
        pltpu.VMEM((240,), jnp.int32),
        pltpu.SemaphoreType.DMA,
    ],
)(_partition_body)


def _layer_body(x_hbm, psrc_hbm, pdst_hbm, pw_hbm, pcnt_hbm, zeros_hbm,
                out_hbm, srcb, dstb, wb, cntb, dstl, rows0, rows1, acc,
                sem0, sem1):
    c = lax.axis_index("c")
    s = lax.axis_index("s")
    pltpu.sync_copy(zeros_hbm, acc.at[pl.ds(s * ZROWS, ZROWS)])

    @pl.when(s == 0)
    def _zero_rem():
        pltpu.sync_copy(zeros_hbm.at[pl.ds(0, ZREM)],
                        acc.at[pl.ds(NS * ZROWS, ZREM)])

    plsc.subcore_barrier()

    def issue(g, rows, sem):
        pltpu.async_copy(x_hbm.at[srcb.at[pl.ds(g * K, K)]], rows, sem)

    def wait(rows, sem):
        pltpu.make_async_copy(x_hbm.at[pl.ds(0, K)], rows, sem).wait()

    def process(g, rows):
        def grp(j, _):
            d = dstb[pl.ds(g * K + j * 16, 16)]
            dl = d - c * HALF
            ok = (dl >= 0) & (dl < HALF)
            dstl[pl.ds(j * 16, 16)] = jnp.where(
                ok, dl, jnp.full((16,), TRASH, jnp.int32) + s)
            wvreg = wb[pl.ds(g * K + j * 16, 16)]
            for e16 in range(16):
                e = j * 16 + e16
                sp = wvreg[e16]
                for k in range(D // 16):
                    sl = pl.ds(k * 16, 16)
                    rows[e, sl] = rows[e, sl] * sp
            return 0

        lax.fori_loop(0, K // 16, grp, 0)
        pltpu.sync_copy(rows, acc.at[dstl], add=True)

    for kk in (0, 1):
        seg = 2 * s + kk
        off = c * NSEG + seg * SEG
        pltpu.sync_copy(psrc_hbm.at[pl.ds(off, SEG)], srcb)
        pltpu.sync_copy(pdst_hbm.at[pl.ds(off, SEG)], dstb)
        pltpu.sync_copy(pw_hbm.at[pl.ds(off, SEG)], wb)
        pltpu.sync_copy(pcnt_hbm.at[pl.ds((c * NW + seg) * 16, 16)], cntb)
        cnt = cntb[pl.ds(0, 16)][0]
        np2 = jnp.maximum((cnt + 2 * K - 1) // (2 * K), 1)

        issue(0, rows0, sem0)
        issue(1, rows1, sem1)

        def pipelined(g2, _):
            g0 = 2 * g2
            wait(rows0, sem0)
            process(g0, rows0)
            issue(g0 + 2, rows0, sem0)
            wait(rows1, sem1)
            process(g0 + 1, rows1)
            issue(g0 + 3, rows1, sem1)
            return 0

        lax.fori_loop(0, np2 - 1, pipelined, 0)
        wait(rows0, sem0)
        process(2 * np2 - 2, rows0)
        wait(rows1, sem1)
        process(2 * np2 - 1, rows1)
    plsc.subcore_barrier()
    pltpu.sync_copy(acc.at[pl.ds(s * OROWS, OROWS)],
                    out_hbm.at[pl.ds(c * HALF + s * OROWS, OROWS)])

    @pl.when(s == 0)
    def _out_rem():
        pltpu.sync_copy(acc.at[pl.ds(NS * OROWS, OREM)],
                        out_hbm.at[pl.ds(c * HALF + NS * OROWS, OREM)])


_layer = functools.partial(
    pl.kernel,
    out_type=jax.ShapeDtypeStruct((N_PAD, D), jnp.float32),
    mesh=_mesh,
    scratch_types=[
        pltpu.VMEM((SEG,), jnp.int32),
        pltpu.VMEM((SEG,), jnp.int32),
        pltpu.VMEM((SEG,), jnp.float32),
        pltpu.VMEM((16,), jnp.int32),
        pltpu.VMEM((K,), jnp.int32),
        pltpu.VMEM((K, D), jnp.float32),
        pltpu.VMEM((K, D), jnp.float32),
        pltpu.VMEM_SHARED((ACC_ROWS, D), jnp.float32),
        pltpu.SemaphoreType.DMA,
        pltpu.SemaphoreType.DMA,
    ],
)(_layer_body)


UPW = B // (NC * NS)


def _gather_users_body(x1, x2, x3, uidx, o1, o2, o3, idxv, rows, sem):
    c = lax.axis_index("c")
    s = lax.axis_index("s")
    base = (s * NC + c) * UPW
    pltpu.sync_copy(uidx.at[pl.ds(base, UPW)], idxv)
    for xh, oh in ((x1, o1), (x2, o2), (x3, o3)):
        pltpu.async_copy(xh.at[idxv], rows, sem).wait()
        pltpu.sync_copy(rows, oh.at[pl.ds(base, UPW)])


_gather_users = functools.partial(
    pl.kernel,
    out_type=(
        jax.ShapeDtypeStruct((B, D), jnp.float32),
        jax.ShapeDtypeStruct((B, D), jnp.float32),
        jax.ShapeDtypeStruct((B, D), jnp.float32),
    ),
    mesh=_mesh,
    scratch_types=[
        pltpu.VMEM((UPW,), jnp.int32),
        pltpu.VMEM((UPW, D), jnp.float32),
        pltpu.SemaphoreType.DMA,
    ],
)(_gather_users_body)


BN = 512


def _scores_body(ua1, ua2, ua3, it1, it2, it3, o_ref):
    dn = (((1,), (1,)), ((), ()))
    acc = lax.dot_general(ua1[...], it1[...], dn,
                          preferred_element_type=jnp.float32)
    acc += lax.dot_general(ua2[...], it2[...], dn,
                           preferred_element_type=jnp.float32)
    acc += lax.dot_general(ua3[...], it3[...], dn,
                           preferred_element_type=jnp.float32)
    o_ref[...] = acc


def _scores(ua1, ua2, ua3, it1, it2, it3):
    grid = (pl.cdiv(NUM_ITEMS, BN),)
    ua_spec = pl.BlockSpec((B, D), lambda j: (0, 0))
    it_spec = pl.BlockSpec((BN, D), lambda j: (j, 0))
    return pl.pallas_call(
        _scores_body,
        grid=grid,
        in_specs=[ua_spec, ua_spec, ua_spec, it_spec, it_spec, it_spec],
        out_specs=pl.BlockSpec((B, BN), lambda j: (0, j)),
        out_shape=jax.ShapeDtypeStruct((B, NUM_ITEMS), jnp.float32),
    )(ua1, ua2, ua3, it1, it2, it3)


def kernel(user_table, item_table, edge_weight, edge_index, user_index):
    src = edge_index[0].astype(jnp.int32)
    dst = edge_index[1].astype(jnp.int32)
    uidx = user_index.astype(jnp.int32)
    w = edge_weight.astype(jnp.float32)
    x0 = jnp.concatenate(
        [user_table, item_table, jnp.zeros((N_PAD - N_NODES, D), jnp.float32)],
        axis=0)
    zeros_in = jnp.zeros((ZROWS, D), jnp.float32)

    psrc, pdst, pw, pcnt = _partition(src, dst, w, jnp.asarray(_STEP))
    x1 = _layer(x0, psrc, pdst, pw, pcnt, zeros_in)
    x2 = _layer(x1, psrc, pdst, pw, pcnt, zeros_in)
    x3 = _layer(x2, psrc, pdst, pw, pcnt, zeros_in)

    ua1, ua2, ua3 = _gather_users(x1, x2, x3, uidx)
    it1 = lax.slice(x1, (NUM_USERS, 0), (N_PAD, D))
    it2 = lax.slice(x2, (NUM_USERS, 0), (N_PAD, D))
    it3 = lax.slice(x3, (NUM_USERS, 0), (N_PAD, D))
    return _scores(ua1, ua2, ua3, it1, it2, it3)


# speedup vs baseline: 4.4996x; 4.4996x over previous
import functools

import jax
import jax.numpy as jnp
import numpy as np
from jax import lax
from jax.experimental import pallas as pl
from jax.experimental.pallas import tpu as pltpu
from jax.experimental.pallas import tpu_sc as plsc

NUM_USERS = 4000
NUM_ITEMS = 6000
N_NODES = NUM_USERS + NUM_ITEMS
D = 128
E = 320000
B = 1024
LAYERS = 3

NC = 2
NS = 16
N_PAD = N_NODES + 16
HALF = N_PAD // NC
ACC_ROWS = HALF + 16
TRASH = HALF
ZROWS = 312
ZREM = ACC_ROWS - NS * ZROWS
OROWS = 312
OREM = HALF - NS * OROWS
EPW = E // NS
K = 80
NCH = EPW // K
NW = NC * NS
PEPW = E // NW
PSEG = 10112

_mesh = plsc.VectorSubcoreMesh(
    core_axis_name="c", subcore_axis_name="s", num_cores=NC, num_subcores=NS
)


_STEP = np.asarray([1 if e > k else 0 for k in range(15) for e in range(16)],
                   np.int32)


def _partition_body(src_hbm, dst_hbm, w_hbm, step_hbm, psrc, pdst, pw, pcnt,
                    isrc, idst, iw, posb0, posb1, posb2, posb3,
                    cntb, stepb, shsrc, shdst, shw, sem):
    c = lax.axis_index("c")
    s = lax.axis_index("s")
    wid = s * NC + c
    base = wid * PEPW
    pltpu.sync_copy(src_hbm.at[pl.ds(base, PEPW)], isrc)
    pltpu.sync_copy(dst_hbm.at[pl.ds(base, PEPW)], idst)
    pltpu.sync_copy(w_hbm.at[pl.ds(base, PEPW)], iw)
    pltpu.sync_copy(step_hbm, stepb)

    iota = lax.iota(jnp.int32, 16)
    sbase = s * PSEG

    posbs = (posb0, posb1, posb2, posb3)

    def compute_issue(cb, pb, lofs, hofs):
        for j in range(K // 16):
            dv = idst[pl.ds(cb + j * 16, 16)]
            islow = -((dv - HALF) >> 31)
            prefix = jnp.zeros((16,), jnp.int32)
            nl = 0
            for k in range(16):
                ilk = -((dv[k] - HALF) >> 31)
                if k < 15:
                    prefix = prefix + stepb[pl.ds(k * 16, 16)] * ilk
                nl = nl + ilk
            poslow = prefix + (sbase + lofs)
            poshigh = (sbase + PEPW - 1 - hofs) - (iota - prefix)
            pos = poshigh + (poslow - poshigh) * islow
            pb[pl.ds(j * 16, 16)] = pos
            lofs = lofs + nl
            hofs = hofs + (16 - nl)
        sl = pl.ds(cb, K)
        pltpu.async_copy(isrc.at[sl], shsrc.at[pb], sem)
        pltpu.async_copy(idst.at[sl], shdst.at[pb], sem)
        pltpu.async_copy(iw.at[sl], shw.at[pb], sem)
        return lofs, hofs

    def drain3():
        for _ in range(3):
            pltpu.make_async_copy(
                isrc.at[pl.ds(0, K)], shsrc.at[pl.ds(0, K)], sem).wait()

    NCHP = PEPW // K
    lofs, hofs = 0, 0
    for i in range(4):
        lofs, hofs = compute_issue(i * K, posbs[i], lofs, hofs)

    def quad(q, carry):
        lofs, hofs = carry
        for i in range(4):
            drain3()
            lofs, hofs = compute_issue((4 + q * 4 + i) * K, posbs[i],
                                       lofs, hofs)
        return (lofs, hofs)

    lofs, hofs = lax.fori_loop(0, (NCHP - 5) // 4, quad, (lofs, hofs))
    drain3()
    lofs, hofs = compute_issue((NCHP - 1) * K, posbs[0], lofs, hofs)
    for _ in range(4):
        drain3()

    obase = wid * PSEG
    pltpu.sync_copy(shsrc.at[pl.ds(sbase, PSEG)], psrc.at[pl.ds(obase, PSEG)])
    pltpu.sync_copy(shdst.at[pl.ds(sbase, PSEG)], pdst.at[pl.ds(obase, PSEG)])
    pltpu.sync_copy(shw.at[pl.ds(sbase, PSEG)], pw.at[pl.ds(obase, PSEG)])
    cntb[pl.ds(0, 16)] = jnp.full((16,), lofs, jnp.int32)
    pltpu.sync_copy(cntb, pcnt.at[pl.ds(wid * 16, 16)])


_partition = functools.partial(
    pl.kernel,
    out_type=(
        jax.ShapeDtypeStruct((NW * PSEG,), jnp.int32),
        jax.ShapeDtypeStruct((NW * PSEG,), jnp.int32),
        jax.ShapeDtypeStruct((NW * PSEG,), jnp.float32),
        jax.ShapeDtypeStruct((NW * 16,), jnp.int32),
    ),
    mesh=_mesh,
    scratch_types=[
        pltpu.VMEM((PEPW,), jnp.int32),
        pltpu.VMEM((PEPW,), jnp.int32),
        pltpu.VMEM((PEPW,), jnp.float32),
        pltpu.VMEM((K,), jnp.int32),
        pltpu.VMEM((K,), jnp.int32),
        pltpu.VMEM((K,), jnp.int32),
        pltpu.VMEM((K,), jnp.int32),
        pltpu.VMEM((16,), jnp.int32),
        pltpu.VMEM((240,), jnp.int32),
        pltpu.VMEM_SHARED((NS * PSEG,), jnp.int32),
        pltpu.VMEM_SHARED((NS * PSEG,), jnp.int32),
        pltpu.VMEM_SHARED((NS * PSEG,), jnp.float32),
        pltpu.SemaphoreType.DMA,
    ],
)(_partition_body)


def _layer_body(x_hbm, psrc_hbm, pdst_hbm, pw_hbm, pcnt_hbm, zeros_hbm,
                out_hbm, srcb, dstb, wb, cntb, wmb, dstl, rows0, rows1, acc,
                sem0, sem1):
    c = lax.axis_index("c")
    s = lax.axis_index("s")
    pltpu.sync_copy(zeros_hbm, acc.at[pl.ds(s * ZROWS, ZROWS)])

    @pl.when(s == 0)
    def _zero_rem():
        pltpu.sync_copy(zeros_hbm.at[pl.ds(0, ZREM)],
                        acc.at[pl.ds(NS * ZROWS, ZREM)])

    plsc.subcore_barrier()

    def issue(g, rows, sem):
        pltpu.async_copy(x_hbm.at[srcb.at[pl.ds(g * K, K)]], rows, sem)

    def wait(rows, sem):
        pltpu.make_async_copy(x_hbm.at[pl.ds(0, K)], rows, sem).wait()

    iota = lax.iota(jnp.int32, 16)

    def process(g, cnt, rows):
        def grp(j, _):
            d = dstb[pl.ds(g * K + j * 16, 16)]
            dl = d - c * HALF
            ok = (dl >= 0) & (dl < HALF)
            dstl[pl.ds(j * 16, 16)] = jnp.where(
                ok, dl, jnp.full((16,), TRASH, jnp.int32) + s)
            idxv = iota + (g * K + j * 16)
            vlt = -((idxv - cnt) >> 31)
            valid = (vlt - c) * (vlt - c)
            wmb[pl.ds(0, 16)] = (wb[pl.ds(g * K + j * 16, 16)]
                                 * valid.astype(jnp.float32))
            wvreg = wmb[pl.ds(0, 16)]
            for e16 in range(16):
                e = j * 16 + e16
                sp = wvreg[e16]
                for k in range(D // 16):
                    sl = pl.ds(k * 16, 16)
                    rows[e, sl] = rows[e, sl] * sp
            return 0

        lax.fori_loop(0, K // 16, grp, 0)
        pltpu.sync_copy(rows, acc.at[dstl], add=True)

    z_i = jnp.zeros((16,), jnp.int32)
    z_f = jnp.zeros((16,), jnp.float32)
    for kk in (0, 1):
        woff = (2 * s + kk) * PSEG
        pltpu.sync_copy(psrc_hbm.at[pl.ds(woff, PEPW)],
                        srcb.at[pl.ds(0, PEPW)])
        pltpu.sync_copy(pdst_hbm.at[pl.ds(woff, PEPW)],
                        dstb.at[pl.ds(0, PEPW)])
        pltpu.sync_copy(pw_hbm.at[pl.ds(woff, PEPW)], wb.at[pl.ds(0, PEPW)])
        for i in range(K // 16):
            srcb[pl.ds(PEPW + i * 16, 16)] = z_i
            wb[pl.ds(PEPW + i * 16, 16)] = z_f
        pltpu.sync_copy(pcnt_hbm.at[pl.ds((2 * s + kk) * 16, 16)], cntb)
        cnt = cntb[pl.ds(0, 16)][0]
        first = c * (cnt // K)
        n_side = (1 - c) * ((cnt + K - 1) // K) + c * (PEPW // K - cnt // K)
        np2 = jnp.maximum((n_side + 1) // 2, 1)

        issue(first, rows0, sem0)
        issue(first + 1, rows1, sem1)

        def pipelined(g2, _):
            g0 = first + 2 * g2
            wait(rows0, sem0)
            process(g0, cnt, rows0)
            issue(g0 + 2, rows0, sem0)
            wait(rows1, sem1)
            process(g0 + 1, cnt, rows1)
            issue(g0 + 3, rows1, sem1)
            return 0

        lax.fori_loop(0, np2 - 1, pipelined, 0)
        wait(rows0, sem0)
        process(first + 2 * np2 - 2, cnt, rows0)
        wait(rows1, sem1)
        process(first + 2 * np2 - 1, cnt, rows1)
    plsc.subcore_barrier()
    pltpu.sync_copy(acc.at[pl.ds(s * OROWS, OROWS)],
                    out_hbm.at[pl.ds(c * HALF + s * OROWS, OROWS)])

    @pl.when(s == 0)
    def _out_rem():
        pltpu.sync_copy(acc.at[pl.ds(NS * OROWS, OREM)],
                        out_hbm.at[pl.ds(c * HALF + NS * OROWS, OREM)])


_layer = functools.partial(
    pl.kernel,
    out_type=jax.ShapeDtypeStruct((N_PAD, D), jnp.float32),
    mesh=_mesh,
    scratch_types=[
        pltpu.VMEM((PEPW + K,), jnp.int32),
        pltpu.VMEM((PEPW + K,), jnp.int32),
        pltpu.VMEM((PEPW + K,), jnp.float32),
        pltpu.VMEM((16,), jnp.int32),
        pltpu.VMEM((16,), jnp.float32),
        pltpu.VMEM((K,), jnp.int32),
        pltpu.VMEM((K, D), jnp.float32),
        pltpu.VMEM((K, D), jnp.float32),
        pltpu.VMEM_SHARED((ACC_ROWS, D), jnp.float32),
        pltpu.SemaphoreType.DMA,
        pltpu.SemaphoreType.DMA,
    ],
)(_layer_body)


UPW = B // (NC * NS)


def _gather_users_body(x1, x2, x3, uidx, o1, o2, o3, idxv, rows, sem):
    c = lax.axis_index("c")
    s = lax.axis_index("s")
    base = (s * NC + c) * UPW
    pltpu.sync_copy(uidx.at[pl.ds(base, UPW)], idxv)
    for xh, oh in ((x1, o1), (x2, o2), (x3, o3)):
        pltpu.async_copy(xh.at[idxv], rows, sem).wait()
        pltpu.sync_copy(rows, oh.at[pl.ds(base, UPW)])


_gather_users = functools.partial(
    pl.kernel,
    out_type=(
        jax.ShapeDtypeStruct((B, D), jnp.float32),
        jax.ShapeDtypeStruct((B, D), jnp.float32),
        jax.ShapeDtypeStruct((B, D), jnp.float32),
    ),
    mesh=_mesh,
    scratch_types=[
        pltpu.VMEM((UPW,), jnp.int32),
        pltpu.VMEM((UPW, D), jnp.float32),
        pltpu.SemaphoreType.DMA,
    ],
)(_gather_users_body)


BN = 512


def _scores_body(ua1, ua2, ua3, it1, it2, it3, o_ref):
    dn = (((1,), (1,)), ((), ()))
    acc = lax.dot_general(ua1[...], it1[...], dn,
                          preferred_element_type=jnp.float32)
    acc += lax.dot_general(ua2[...], it2[...], dn,
                           preferred_element_type=jnp.float32)
    acc += lax.dot_general(ua3[...], it3[...], dn,
                           preferred_element_type=jnp.float32)
    o_ref[...] = acc


def _scores(ua1, ua2, ua3, it1, it2, it3):
    grid = (pl.cdiv(NUM_ITEMS, BN),)
    ua_spec = pl.BlockSpec((B, D), lambda j: (0, 0))
    it_spec = pl.BlockSpec((BN, D), lambda j: (j, 0))
    return pl.pallas_call(
        _scores_body,
        grid=grid,
        in_specs=[ua_spec, ua_spec, ua_spec, it_spec, it_spec, it_spec],
        out_specs=pl.BlockSpec((B, BN), lambda j: (0, j)),
        out_shape=jax.ShapeDtypeStruct((B, NUM_ITEMS), jnp.float32),
    )(ua1, ua2, ua3, it1, it2, it3)


def kernel(user_table, item_table, edge_weight, edge_index, user_index):
    src = edge_index[0].astype(jnp.int32)
    dst = edge_index[1].astype(jnp.int32)
    uidx = user_index.astype(jnp.int32)
    w = edge_weight.astype(jnp.float32)
    x0 = jnp.concatenate(
        [user_table, item_table, jnp.zeros((N_PAD - N_NODES, D), jnp.float32)],
        axis=0)
    zeros_in = jnp.zeros((ZROWS, D), jnp.float32)

    psrc, pdst, pw, pcnt = _partition(src, dst, w, jnp.asarray(_STEP))
    x1 = _layer(x0, psrc, pdst, pw, pcnt, zeros_in)
    x2 = _layer(x1, psrc, pdst, pw, pcnt, zeros_in)
    x3 = _layer(x2, psrc, pdst, pw, pcnt, zeros_in)

    ua1, ua2, ua3 = _gather_users(x1, x2, x3, uidx)
    it1 = lax.slice(x1, (NUM_USERS, 0), (N_PAD, D))
    it2 = lax.slice(x2, (NUM_USERS, 0), (N_PAD, D))
    it3 = lax.slice(x3, (NUM_USERS, 0), (N_PAD, D))
    return _scores(ua1, ua2, ua3, it1, it2, it3)
